# baseline (speedup 1.0000x reference)
import functools

import jax
import jax.numpy as jnp
from jax.experimental import pallas as pl
from jax.experimental.pallas import tpu as pltpu


def _aug(pts, scale, norm_first):
    bf = jnp.bfloat16
    p = pts.shape[0]
    sq = pts[:, 0:1] ** 2 + pts[:, 1:2] ** 2 + pts[:, 2:3] ** 2
    hi = sq.astype(bf)
    lo = (sq - hi.astype(jnp.float32)).astype(bf)
    ones = jnp.ones((p, 2), bf)
    zero = jnp.zeros((p, 1), bf)
    norm = [hi, lo, ones] if norm_first else [ones, hi, lo]
    return jnp.concatenate([(scale * pts).astype(bf)] + norm + [zero], axis=1)


def _chamfer_batch_kernel(x1_ref, x2_ref, d1_ref, d2_ref, *, mc):
    m = x2_ref.shape[1]
    a_aug = _aug(x1_ref[0], 1.0, norm_first=True)
    b_aug = _aug(x2_ref[0], -2.0, norm_first=False)

    dims = (((1,), (1,)), ((), ()))
    rowacc = None
    colmins = []
    for mi in range(m // mc):
        d = jax.lax.dot_general(
            a_aug, b_aug[mi * mc:(mi + 1) * mc, :], dims,
            preferred_element_type=jnp.float32)
        rowacc = d if rowacc is None else jnp.minimum(rowacc, d)
        colmins.append(jnp.min(d, axis=0))
    d1_ref[0, :, 0] = jnp.min(rowacc, axis=1)
    d2_ref[0, 0, :] = jnp.concatenate(colmins)


@functools.partial(jax.jit, static_argnames=("mc",))
def _chamfer(input1, input2, mc=1024):
    b, n, _ = input1.shape
    m = input2.shape[1]
    return pl.pallas_call(
        functools.partial(_chamfer_batch_kernel, mc=mc),
        grid=(b,),
        in_specs=[
            pl.BlockSpec((1, n, 3), lambda bi: (bi, 0, 0)),
            pl.BlockSpec((1, m, 3), lambda bi: (bi, 0, 0)),
        ],
        out_specs=[
            pl.BlockSpec((1, n, 1), lambda bi: (bi, 0, 0)),
            pl.BlockSpec((1, 1, m), lambda bi: (bi, 0, 0)),
        ],
        out_shape=[
            jax.ShapeDtypeStruct((b, n, 1), jnp.float32),
            jax.ShapeDtypeStruct((b, 1, m), jnp.float32),
        ],
    )(input1, input2)


def kernel(input1, input2):
    dist1, dist2 = _chamfer(input1, input2)
    return (dist1[:, :, 0], dist2[:, 0, :])


# speedup vs baseline: 1.0748x; 1.0748x over previous
import functools

import jax
import jax.numpy as jnp
from jax.experimental import pallas as pl
from jax.experimental.pallas import tpu as pltpu


def _chamfer_batch_kernel(x1_ref, x2tn_ref, d1_ref, d2_ref, b_scr, *, mc):
    bf = jnp.bfloat16
    x1 = x1_ref[0]
    x2tn = x2tn_ref[0]
    n = x1.shape[0]
    m = x2tn.shape[1]

    x2sq = 0.25 * (x2tn[0:1, :] ** 2 + x2tn[1:2, :] ** 2 + x2tn[2:3, :] ** 2)
    h2 = x2sq.astype(bf)
    r = x2sq - h2.astype(jnp.float32)
    m2 = r.astype(bf)
    l2 = (r - m2.astype(jnp.float32)).astype(bf)
    b_scr[0:3, :] = x2tn.astype(bf)
    b_scr[3:5, :] = jnp.ones((2, m), bf)
    b_scr[5:6, :] = h2
    b_scr[6:7, :] = m2
    b_scr[7:8, :] = l2

    x1sq = x1[:, 0:1] ** 2 + x1[:, 1:2] ** 2 + x1[:, 2:3] ** 2
    h1 = x1sq.astype(bf)
    l1 = (x1sq - h1.astype(jnp.float32)).astype(bf)
    a_aug = jnp.concatenate(
        [x1.astype(bf), h1, l1, jnp.ones((n, 3), bf)], axis=1)

    rowacc = None
    colmins = []
    for mi in range(m // mc):
        d = jnp.dot(a_aug, b_scr[:, mi * mc:(mi + 1) * mc],
                    preferred_element_type=jnp.float32)
        rowacc = d if rowacc is None else jnp.minimum(rowacc, d)
        colmins.append(jnp.min(d, axis=0))
    d1_ref[0, :, 0] = jnp.min(rowacc, axis=1)
    d2_ref[0, 0, :] = jnp.concatenate(colmins)


@functools.partial(jax.jit, static_argnames=("mc",))
def _chamfer(input1, input2, mc=1024):
    b, n, _ = input1.shape
    m = input2.shape[1]
    x2t = -2.0 * jnp.transpose(input2, (0, 2, 1))
    return pl.pallas_call(
        functools.partial(_chamfer_batch_kernel, mc=mc),
        grid=(b,),
        in_specs=[
            pl.BlockSpec((1, n, 3), lambda bi: (bi, 0, 0)),
            pl.BlockSpec((1, 3, m), lambda bi: (bi, 0, 0)),
        ],
        out_specs=[
            pl.BlockSpec((1, n, 1), lambda bi: (bi, 0, 0)),
            pl.BlockSpec((1, 1, m), lambda bi: (bi, 0, 0)),
        ],
        out_shape=[
            jax.ShapeDtypeStruct((b, n, 1), jnp.float32),
            jax.ShapeDtypeStruct((b, 1, m), jnp.float32),
        ],
        scratch_shapes=[pltpu.VMEM((8, m), jnp.bfloat16)],
    )(input1, x2t)


def kernel(input1, input2):
    dist1, dist2 = _chamfer(input1, input2)
    return (dist1[:, :, 0], dist2[:, 0, :])


# speedup vs baseline: 1.0756x; 1.0007x over previous
import functools

import jax
import jax.numpy as jnp
from jax.experimental import pallas as pl
from jax.experimental.pallas import tpu as pltpu


def _chamfer_batch_kernel(x1_ref, x2tn_ref, d1_ref, d2_ref, b_scr, *, mc):
    bf = jnp.bfloat16
    x1 = x1_ref[0]
    x2tn = x2tn_ref[0]
    n = x1.shape[0]
    m = x2tn.shape[1]

    x2sq = 0.25 * (x2tn[0:1, :] ** 2 + x2tn[1:2, :] ** 2 + x2tn[2:3, :] ** 2)
    h2 = x2sq.astype(bf)
    r = x2sq - h2.astype(jnp.float32)
    m2 = r.astype(bf)
    l2 = (r - m2.astype(jnp.float32)).astype(bf)
    b_scr[0:3, :] = x2tn.astype(bf)
    b_scr[3:5, :] = jnp.ones((2, m), bf)
    b_scr[5:6, :] = h2
    b_scr[6:7, :] = m2
    b_scr[7:8, :] = l2

    x1sq = x1[:, 0:1] ** 2 + x1[:, 1:2] ** 2 + x1[:, 2:3] ** 2
    h1 = x1sq.astype(bf)
    l1 = (x1sq - h1.astype(jnp.float32)).astype(bf)
    a_aug = jnp.concatenate(
        [x1.astype(bf), h1, l1, jnp.ones((n, 3), bf)], axis=1)

    rowacc = None
    colmins = []
    for mi in range(m // mc):
        d = jnp.dot(a_aug, b_scr[:, mi * mc:(mi + 1) * mc],
                    preferred_element_type=jnp.float32)
        rowacc = d if rowacc is None else jnp.minimum(rowacc, d)
        colmins.append(jnp.min(d, axis=0))
    d1_ref[0, :, 0] = jnp.min(rowacc, axis=1)
    d2_ref[0, 0, :] = jnp.concatenate(colmins)


@functools.partial(jax.jit, static_argnames=("mc",))
def _chamfer(input1, input2, mc=2048):
    b, n, _ = input1.shape
    m = input2.shape[1]
    x2t = -2.0 * jnp.transpose(input2, (0, 2, 1))
    return pl.pallas_call(
        functools.partial(_chamfer_batch_kernel, mc=mc),
        grid=(b,),
        in_specs=[
            pl.BlockSpec((1, n, 3), lambda bi: (bi, 0, 0)),
            pl.BlockSpec((1, 3, m), lambda bi: (bi, 0, 0)),
        ],
        out_specs=[
            pl.BlockSpec((1, n, 1), lambda bi: (bi, 0, 0)),
            pl.BlockSpec((1, 1, m), lambda bi: (bi, 0, 0)),
        ],
        out_shape=[
            jax.ShapeDtypeStruct((b, n, 1), jnp.float32),
            jax.ShapeDtypeStruct((b, 1, m), jnp.float32),
        ],
        scratch_shapes=[pltpu.VMEM((8, m), jnp.bfloat16)],
    )(input1, x2t)


def kernel(input1, input2):
    dist1, dist2 = _chamfer(input1, input2)
    return (dist1[:, :, 0], dist2[:, 0, :])
